# Initial kernel scaffold; baseline (speedup 1.0000x reference)
import jax
import jax.numpy as jnp
from jax.experimental import pallas as pl


def kernel(x, W):
    raise NotImplementedError("write your pallas kernel here")


# speedup vs baseline: 1.1226x; 1.1226x over previous
import jax
import jax.numpy as jnp
from jax.experimental import pallas as pl

D_MODEL = 4096
N_EXP = 64
TOPK = 8
BT = 512


def _router_body(x_ref, w_ref, gs_ref, idx_ref, tw_ref):
    xb = x_ref[...]
    wb = w_ref[...]
    logits = jax.lax.dot_general(
        xb, wb, (((1,), (1,)), ((), ())), preferred_element_type=jnp.float32
    )
    m = jnp.max(logits, axis=-1, keepdims=True)
    e = jnp.exp(logits - m)
    s = jnp.sum(e, axis=-1, keepdims=True)
    gs = e / s
    gs_ref[...] = gs

    iota = jax.lax.broadcasted_iota(jnp.int32, gs.shape, 1)
    work = gs
    ws, ids = [], []
    for _ in range(TOPK):
        mj = jnp.max(work, axis=-1, keepdims=True)
        is_max = work == mj
        ij = jnp.min(jnp.where(is_max, iota, N_EXP), axis=-1, keepdims=True)
        ws.append(mj)
        ids.append(ij)
        work = jnp.where(iota == ij, -jnp.inf, work)
    w8 = jnp.concatenate(ws, axis=1)
    i8 = jnp.concatenate(ids, axis=1)
    tw_ref[...] = w8 / (jnp.sum(w8, axis=1, keepdims=True) + 1e-8)
    idx_ref[...] = i8


def kernel(x, W):
    if x.ndim == 3:
        x = x.mean(axis=1)
    B = x.shape[0]
    out = pl.pallas_call(
        _router_body,
        grid=(B // BT,),
        in_specs=[
            pl.BlockSpec((BT, D_MODEL), lambda i: (i, 0)),
            pl.BlockSpec((N_EXP, D_MODEL), lambda i: (0, 0)),
        ],
        out_specs=[
            pl.BlockSpec((BT, N_EXP), lambda i: (i, 0)),
            pl.BlockSpec((BT, TOPK), lambda i: (i, 0)),
            pl.BlockSpec((BT, TOPK), lambda i: (i, 0)),
        ],
        out_shape=[
            jax.ShapeDtypeStruct((B, N_EXP), jnp.float32),
            jax.ShapeDtypeStruct((B, TOPK), jnp.int32),
            jax.ShapeDtypeStruct((B, TOPK), jnp.float32),
        ],
    )(x, W)
    return (out[0], out[1], out[2])


# speedup vs baseline: 1.2916x; 1.1505x over previous
import jax
import jax.numpy as jnp
from jax.experimental import pallas as pl

D_MODEL = 4096
N_EXP = 64
TOPK = 8
BT = 1024


def _router_body(x_ref, w_ref, gs_ref, idx_ref, tw_ref):
    xb = x_ref[...]
    wb = w_ref[...]
    logits = jax.lax.dot_general(
        xb, wb, (((1,), (1,)), ((), ())), preferred_element_type=jnp.float32
    )
    m = jnp.max(logits, axis=-1, keepdims=True)
    e = jnp.exp(logits - m)
    s = jnp.sum(e, axis=-1, keepdims=True)
    gs = e / s
    gs_ref[...] = gs

    iota = jax.lax.broadcasted_iota(jnp.int32, gs.shape, 1)
    work = gs
    ws, ids = [], []
    for _ in range(TOPK):
        mj = jnp.max(work, axis=-1, keepdims=True)
        is_max = work == mj
        ij = jnp.min(jnp.where(is_max, iota, N_EXP), axis=-1, keepdims=True)
        ws.append(mj)
        ids.append(ij)
        work = jnp.where(iota == ij, -jnp.inf, work)
    w8 = jnp.concatenate(ws, axis=1)
    i8 = jnp.concatenate(ids, axis=1)
    tw_ref[...] = w8 / (jnp.sum(w8, axis=1, keepdims=True) + 1e-8)
    idx_ref[...] = i8


def kernel(x, W):
    if x.ndim == 3:
        x = x.mean(axis=1)
    B = x.shape[0]
    out = pl.pallas_call(
        _router_body,
        grid=(B // BT,),
        in_specs=[
            pl.BlockSpec((BT, D_MODEL), lambda i: (i, 0)),
            pl.BlockSpec((N_EXP, D_MODEL), lambda i: (0, 0)),
        ],
        out_specs=[
            pl.BlockSpec((BT, N_EXP), lambda i: (i, 0)),
            pl.BlockSpec((BT, TOPK), lambda i: (i, 0)),
            pl.BlockSpec((BT, TOPK), lambda i: (i, 0)),
        ],
        out_shape=[
            jax.ShapeDtypeStruct((B, N_EXP), jnp.float32),
            jax.ShapeDtypeStruct((B, TOPK), jnp.int32),
            jax.ShapeDtypeStruct((B, TOPK), jnp.float32),
        ],
    )(x, W)
    return (out[0], out[1], out[2])
